# Initial kernel scaffold; baseline (speedup 1.0000x reference)
import functools

import jax
import jax.numpy as jnp
from jax import lax
from jax.experimental import pallas as pl
from jax.experimental.pallas import tpu as pltpu
from jax.experimental.pallas import tpu_sc as plsc

N = 10000
NP = 10240
E = 160000
D_IN = 256
HID = 64
HEADS = 8
OUT = 256

BN = 1024
NB = NP // BN
EB = 80
RP = NP // 16


def _tc1_body(x_ref, ws_ref, as_ref, wd_ref, ad_ref, hs_ref, als_ref, ald_ref):
    x = x_ref[0]
    for h in range(HEADS):
        ws_h = ws_ref[0, :, h * HID:(h + 1) * HID]
        yh = jnp.dot(x, ws_h, preferred_element_type=jnp.float32)
        hs_ref[0, h] = yh
        als_ref[0, h] = (yh * as_ref[0, h][None, :]).sum(-1)
        wd_h = wd_ref[0, :, h * HID:(h + 1) * HID]
        hdh = jnp.dot(x, wd_h, preferred_element_type=jnp.float32)
        ald_ref[0, h] = (hdh * ad_ref[0, h][None, :]).sum(-1)


def _tc1(x_pack, ws_pack, as_pack, wd_pack, ad_pack):
    return pl.pallas_call(
        _tc1_body,
        grid=(2, NB),
        in_specs=[
            pl.BlockSpec((1, BN, D_IN), lambda t, nb: (t, nb, 0)),
            pl.BlockSpec((1, D_IN, HEADS * HID), lambda t, nb: (t, 0, 0)),
            pl.BlockSpec((1, HEADS, HID), lambda t, nb: (t, 0, 0)),
            pl.BlockSpec((1, D_IN, HEADS * HID), lambda t, nb: (1 - t, 0, 0)),
            pl.BlockSpec((1, HEADS, HID), lambda t, nb: (1 - t, 0, 0)),
        ],
        out_specs=[
            pl.BlockSpec((1, HEADS, BN, HID), lambda t, nb: (t, 0, nb, 0)),
            pl.BlockSpec((1, HEADS, BN), lambda t, nb: (t, 0, nb)),
            pl.BlockSpec((1, HEADS, BN), lambda t, nb: (1 - t, 0, nb)),
        ],
        out_shape=[
            jax.ShapeDtypeStruct((2, HEADS, NP, HID), jnp.float32),
            jax.ShapeDtypeStruct((2, HEADS, NP), jnp.float32),
            jax.ShapeDtypeStruct((2, HEADS, NP), jnp.float32),
        ],
    )(x_pack, ws_pack, as_pack, wd_pack, ad_pack)


def _tc2_body(x_ref, w2a_ref, w2b_ref, a2s_ref, a2d_ref,
              hs2_ref, als2_ref, ald2_ref):
    hs2 = jnp.zeros((BN, HID), jnp.float32)
    hd2 = jnp.zeros((BN, HID), jnp.float32)
    for h in range(HEADS):
        xh = x_ref[0, h]
        hs2 = hs2 + jnp.dot(xh, w2a_ref[0, h * HID:(h + 1) * HID, :],
                            preferred_element_type=jnp.float32)
        hd2 = hd2 + jnp.dot(xh, w2b_ref[0, h * HID:(h + 1) * HID, :],
                            preferred_element_type=jnp.float32)
    hs2_ref[0] = hs2
    als2_ref[0] = (hs2 * a2s_ref[0][None, :]).sum(-1)
    ald2_ref[0] = (hd2 * a2d_ref[0][None, :]).sum(-1)


def _tc2(out1, w2_pack, a2s_pack, a2d_pack):
    return pl.pallas_call(
        _tc2_body,
        grid=(2, NB),
        in_specs=[
            pl.BlockSpec((1, HEADS, BN, HID), lambda u, nb: (1 - u, 0, nb, 0)),
            pl.BlockSpec((1, HEADS * HID, HID), lambda u, nb: (u, 0, 0)),
            pl.BlockSpec((1, HEADS * HID, HID), lambda u, nb: (1 - u, 0, 0)),
            pl.BlockSpec((1, HID), lambda u, nb: (u, 0)),
            pl.BlockSpec((1, HID), lambda u, nb: (1 - u, 0)),
        ],
        out_specs=[
            pl.BlockSpec((1, BN, HID), lambda u, nb: (u, nb, 0)),
            pl.BlockSpec((1, BN), lambda u, nb: (u, nb)),
            pl.BlockSpec((1, BN), lambda u, nb: (1 - u, nb)),
        ],
        out_shape=[
            jax.ShapeDtypeStruct((2, NP, HID), jnp.float32),
            jax.ShapeDtypeStruct((2, NP), jnp.float32),
            jax.ShapeDtypeStruct((2, NP), jnp.float32),
        ],
    )(out1, w2_pack, w2_pack, a2s_pack, a2d_pack)


def _tc3_body(x_ref, w_ref, b_ref, o_ref):
    o_ref[0] = (jnp.dot(x_ref[0], w_ref[...], preferred_element_type=jnp.float32)
                + b_ref[...][None, :])


def _tc3(out2, lin_w, lin_b):
    return pl.pallas_call(
        _tc3_body,
        grid=(2, NB),
        in_specs=[
            pl.BlockSpec((1, BN, HID), lambda t, nb: (t, nb, 0)),
            pl.BlockSpec((HID, OUT), lambda t, nb: (0, 0)),
            pl.BlockSpec((OUT,), lambda t, nb: (0,)),
        ],
        out_specs=pl.BlockSpec((1, BN, OUT), lambda t, nb: (t, nb, 0)),
        out_shape=jax.ShapeDtypeStruct((2, NP, OUT), jnp.float32),
    )(out2, lin_w, lin_b)


def _sc_gat_body(H, src_hbm, dst_hbm, als_hbm, ald_hbm, hs_hbm, b_hbm,
                 out_hbm, als_v, ald_v, srcb, dstb, idxb, exb, rows_v,
                 acc_v, denb_v, bias_v, num_sh, den_sh, sem):
    c = lax.axis_index("c")
    s = lax.axis_index("s")
    zero16 = jnp.zeros((16,), jnp.float32)
    eper = E // 16

    def phase(h, _):
        tb = (c * H + h) * NP
        pltpu.sync_copy(als_hbm.at[pl.ds(tb, NP)], als_v)
        pltpu.sync_copy(ald_hbm.at[pl.ds(tb, NP)], ald_v)
        pltpu.sync_copy(b_hbm.at[pl.ds((c * H + h) * HID, HID)], bias_v)

        def zrow(r, _):
            for q in range(4):
                acc_v[r, pl.ds(q * 16, 16)] = zero16
            return ()
        lax.fori_loop(0, RP, zrow, ())

        def zden(i, _):
            denb_v[pl.ds(i * 16, 16)] = zero16
            return ()
        lax.fori_loop(0, RP // 16, zden, ())
        pltpu.sync_copy(acc_v, num_sh.at[pl.ds(s * RP, RP)])
        pltpu.sync_copy(denb_v, den_sh.at[pl.ds(s * RP, RP)])
        plsc.subcore_barrier()

        ebase0 = c * E + s * eper

        def eblk(i, _):
            eb = ebase0 + i * EB
            pltpu.sync_copy(src_hbm.at[pl.ds(eb, EB)], srcb)
            pltpu.sync_copy(dst_hbm.at[pl.ds(eb, EB)], dstb)
            for j in range(EB // 16):
                sl = pl.ds(j * 16, 16)
                sv = srcb[sl]
                dv = dstb[sl]
                idxb[sl] = sv + tb
                av = plsc.load_gather(als_v, [sv])
                ad = plsc.load_gather(ald_v, [dv])
                e = av + ad
                e = jnp.where(e >= 0.0, e, 0.2 * e)
                exb[sl] = jnp.exp(e)
            pltpu.async_copy(hs_hbm.at[idxb], rows_v, sem).wait()
            for el in range(EB):
                exs = plsc.load_gather(exb, [jnp.full((16,), el, jnp.int32)])
                for q in range(4):
                    qsl = pl.ds(q * 16, 16)
                    rows_v[el, qsl] = rows_v[el, qsl] * exs
            pltpu.sync_copy(rows_v, num_sh.at[dstb], add=True)
            pltpu.sync_copy(exb, den_sh.at[dstb], add=True)
            return ()
        lax.fori_loop(0, eper // EB, eblk, ())
        plsc.subcore_barrier()

        rb = s * RP
        pltpu.sync_copy(num_sh.at[pl.ds(rb, RP)], acc_v)
        pltpu.sync_copy(den_sh.at[pl.ds(rb, RP)], denb_v)
        bvs = [bias_v[pl.ds(q * 16, 16)] for q in range(4)]

        def drow(rc, _):
            for rr in range(16):
                r = rc * 16 + rr
                dsp = plsc.load_gather(denb_v, [jnp.broadcast_to(r, (16,))])
                dsp = dsp + 1e-16
                for q in range(4):
                    qsl = pl.ds(q * 16, 16)
                    acc_v[r, qsl] = acc_v[r, qsl] / dsp + bvs[q]
            return ()
        lax.fori_loop(0, RP // 16, drow, ())
        pltpu.sync_copy(acc_v, out_hbm.at[pl.ds(tb + rb, RP)])
        plsc.subcore_barrier()
        return ()

    lax.fori_loop(0, H, phase, ())


def _sc_gat(H, src_all, dst_all, als_flat, ald_flat, hs_flat, b_flat):
    mesh = plsc.VectorSubcoreMesh(core_axis_name="c", subcore_axis_name="s")
    f = pl.kernel(
        functools.partial(_sc_gat_body, H),
        out_type=jax.ShapeDtypeStruct((2 * H * NP, HID), jnp.float32),
        mesh=mesh,
        scratch_types=[
            pltpu.VMEM((NP,), jnp.float32),
            pltpu.VMEM((NP,), jnp.float32),
            pltpu.VMEM((EB,), jnp.int32),
            pltpu.VMEM((EB,), jnp.int32),
            pltpu.VMEM((EB,), jnp.int32),
            pltpu.VMEM((EB,), jnp.float32),
            pltpu.VMEM((EB, HID), jnp.float32),
            pltpu.VMEM((RP, HID), jnp.float32),
            pltpu.VMEM((RP,), jnp.float32),
            pltpu.VMEM((HID,), jnp.float32),
            pltpu.VMEM_SHARED((NP, HID), jnp.float32),
            pltpu.VMEM_SHARED((NP,), jnp.float32),
            pltpu.SemaphoreType.DMA,
        ],
    )
    return f(src_all, dst_all, als_flat, ald_flat, hs_flat, b_flat)


def kernel(x_author, x_paper, ei_writes, ei_written_by,
           w1s_wr, w1d_wr, a1s_wr, a1d_wr, b1_wr,
           w1s_wb, w1d_wb, a1s_wb, a1d_wb, b1_wb,
           w2_wr, a2s_wr, a2d_wr, b2_wr,
           w2_wb, a2s_wb, a2d_wb, b2_wb,
           lin_w, lin_b):
    f32 = jnp.float32
    x_pack = jnp.zeros((2, NP, D_IN), f32)
    x_pack = x_pack.at[0, :N].set(x_author).at[1, :N].set(x_paper)
    ws_pack = jnp.stack([w1s_wr, w1s_wb])
    as_pack = jnp.stack([a1s_wr, a1s_wb])
    wd_pack = jnp.stack([w1d_wb, w1d_wr])
    ad_pack = jnp.stack([a1d_wb, a1d_wr])
    src_all = jnp.concatenate([ei_writes[0], ei_written_by[0]])
    dst_all = jnp.concatenate([ei_writes[1], ei_written_by[1]])
    b1_flat = jnp.concatenate([b1_wr, b1_wb])
    b2_flat = jnp.concatenate([b2_wr, b2_wb])
    w2_pack = jnp.stack([w2_wr, w2_wb])
    a2s_pack = jnp.stack([a2s_wr[0], a2s_wb[0]])
    a2d_pack = jnp.stack([a2d_wr[0], a2d_wb[0]])

    hs_all, als_all, ald_all = _tc1(x_pack, ws_pack, as_pack, wd_pack, ad_pack)
    out1 = _sc_gat(HEADS, src_all, dst_all,
                   als_all.reshape(-1), ald_all.reshape(-1),
                   hs_all.reshape(-1, HID), b1_flat)
    out1 = out1.reshape(2, HEADS, NP, HID)

    hs2, als2, ald2 = _tc2(out1, w2_pack, a2s_pack, a2d_pack)
    out2 = _sc_gat(1, src_all, dst_all,
                   als2.reshape(-1), ald2.reshape(-1),
                   hs2.reshape(-1, HID), b2_flat)

    res = _tc3(out2.reshape(2, NP, HID), lin_w, lin_b)
    return (res[1, :N], res[0, :N])


# speedup vs baseline: 15.7780x; 15.7780x over previous
import functools

import jax
import jax.numpy as jnp
from jax import lax
from jax.experimental import pallas as pl
from jax.experimental.pallas import tpu as pltpu
from jax.experimental.pallas import tpu_sc as plsc

N = 10000
NP = 10240
E = 160000
D_IN = 256
HID = 64
HEADS = 8
OUT = 256

BN = 1024
NB = NP // BN
EB = 80
RP = NP // 16


def _tc1_body(x_ref, ws_ref, as_ref, wd_ref, ad_ref, hs_ref, als_ref, ald_ref):
    x = x_ref[0]
    for h in range(HEADS):
        ws_h = ws_ref[0, :, h * HID:(h + 1) * HID]
        yh = jnp.dot(x, ws_h, preferred_element_type=jnp.float32)
        hs_ref[0, h] = yh
        als_ref[0, h] = (yh * as_ref[0, h][None, :]).sum(-1)
        wd_h = wd_ref[0, :, h * HID:(h + 1) * HID]
        hdh = jnp.dot(x, wd_h, preferred_element_type=jnp.float32)
        ald_ref[0, h] = (hdh * ad_ref[0, h][None, :]).sum(-1)


def _tc1(x_pack, ws_pack, as_pack, wd_pack, ad_pack):
    return pl.pallas_call(
        _tc1_body,
        grid=(2, NB),
        in_specs=[
            pl.BlockSpec((1, BN, D_IN), lambda t, nb: (t, nb, 0)),
            pl.BlockSpec((1, D_IN, HEADS * HID), lambda t, nb: (t, 0, 0)),
            pl.BlockSpec((1, HEADS, HID), lambda t, nb: (t, 0, 0)),
            pl.BlockSpec((1, D_IN, HEADS * HID), lambda t, nb: (t, 0, 0)),
            pl.BlockSpec((1, HEADS, HID), lambda t, nb: (t, 0, 0)),
        ],
        out_specs=[
            pl.BlockSpec((1, HEADS, BN, HID), lambda t, nb: (t, 0, nb, 0)),
            pl.BlockSpec((1, HEADS, BN), lambda t, nb: (t, 0, nb)),
            pl.BlockSpec((1, HEADS, BN), lambda t, nb: (1 - t, 0, nb)),
        ],
        out_shape=[
            jax.ShapeDtypeStruct((2, HEADS, NP, HID), jnp.float32),
            jax.ShapeDtypeStruct((2, HEADS, NP), jnp.float32),
            jax.ShapeDtypeStruct((2, HEADS, NP), jnp.float32),
        ],
    )(x_pack, ws_pack, as_pack, wd_pack, ad_pack)


def _tc2_body(x_ref, w2a_ref, w2b_ref, a2s_ref, a2d_ref,
              hs2_ref, als2_ref, ald2_ref):
    hs2 = jnp.zeros((BN, HID), jnp.float32)
    hd2 = jnp.zeros((BN, HID), jnp.float32)
    for h in range(HEADS):
        xh = x_ref[0, h]
        hs2 = hs2 + jnp.dot(xh, w2a_ref[0, h * HID:(h + 1) * HID, :],
                            preferred_element_type=jnp.float32)
        hd2 = hd2 + jnp.dot(xh, w2b_ref[0, h * HID:(h + 1) * HID, :],
                            preferred_element_type=jnp.float32)
    hs2_ref[0] = hs2
    als2_ref[0, 0] = (hs2 * a2s_ref[0, 0][None, :]).sum(-1)
    ald2_ref[0, 0] = (hd2 * a2d_ref[0, 0][None, :]).sum(-1)


def _tc2(out1, w2_pack, a2s_pack, a2d_pack):
    return pl.pallas_call(
        _tc2_body,
        grid=(2, NB),
        in_specs=[
            pl.BlockSpec((1, HEADS, BN, HID), lambda u, nb: (1 - u, 0, nb, 0)),
            pl.BlockSpec((1, HEADS * HID, HID), lambda u, nb: (u, 0, 0)),
            pl.BlockSpec((1, HEADS * HID, HID), lambda u, nb: (1 - u, 0, 0)),
            pl.BlockSpec((1, 1, HID), lambda u, nb: (u, 0, 0)),
            pl.BlockSpec((1, 1, HID), lambda u, nb: (1 - u, 0, 0)),
        ],
        out_specs=[
            pl.BlockSpec((1, BN, HID), lambda u, nb: (u, nb, 0)),
            pl.BlockSpec((1, 1, BN), lambda u, nb: (u, 0, nb)),
            pl.BlockSpec((1, 1, BN), lambda u, nb: (1 - u, 0, nb)),
        ],
        out_shape=[
            jax.ShapeDtypeStruct((2, NP, HID), jnp.float32),
            jax.ShapeDtypeStruct((2, 1, NP), jnp.float32),
            jax.ShapeDtypeStruct((2, 1, NP), jnp.float32),
        ],
    )(out1, w2_pack, w2_pack, a2s_pack, a2d_pack)


def _tc3_body(x_ref, w_ref, b_ref, o_ref):
    o_ref[0] = (jnp.dot(x_ref[0], w_ref[...], preferred_element_type=jnp.float32)
                + b_ref[...][None, :])


def _tc3(out2, lin_w, lin_b):
    return pl.pallas_call(
        _tc3_body,
        grid=(2, NB),
        in_specs=[
            pl.BlockSpec((1, BN, HID), lambda t, nb: (t, nb, 0)),
            pl.BlockSpec((HID, OUT), lambda t, nb: (0, 0)),
            pl.BlockSpec((OUT,), lambda t, nb: (0,)),
        ],
        out_specs=pl.BlockSpec((1, BN, OUT), lambda t, nb: (t, nb, 0)),
        out_shape=jax.ShapeDtypeStruct((2, NP, OUT), jnp.float32),
    )(out2, lin_w, lin_b)


def _vgather(x, idx16):
    dnums = lax.GatherDimensionNumbers(
        offset_dims=(), collapsed_slice_dims=(0,), start_index_map=(0,))
    return lax.gather(x, idx16[:, None], dnums, (1,),
                      mode=lax.GatherScatterMode.PROMISE_IN_BOUNDS)


def _sc_gat_body(H, src_hbm, dst_hbm, als_hbm, ald_hbm, hs_hbm, b_hbm,
                 out_hbm, als_v, ald_v, srcb, dstb, exb, rows_v,
                 acc_v, denb_v, bias_v, num_sh, den_sh, sem):
    c = lax.axis_index("c")
    s = lax.axis_index("s")
    zero16 = jnp.zeros((16,), jnp.float32)
    eper = E // 16

    def phase(h, _):
        tb = (c * H + h) * NP
        pltpu.sync_copy(als_hbm.at[pl.ds(tb, NP)], als_v)
        pltpu.sync_copy(ald_hbm.at[pl.ds(tb, NP)], ald_v)
        pltpu.sync_copy(b_hbm.at[pl.ds((c * H + h) * HID, HID)], bias_v)

        def zacc(r, _):
            for q in range(4):
                acc_v[r, pl.ds(q * 16, 16)] = zero16
            return ()
        lax.fori_loop(0, RP, zacc, ())

        def zden(i, _):
            denb_v[pl.ds(i * 16, 16)] = zero16
            return ()
        lax.fori_loop(0, RP // 16, zden, ())
        pltpu.sync_copy(acc_v, num_sh.at[pl.ds(s * RP, RP)])
        pltpu.sync_copy(denb_v, den_sh.at[pl.ds(s * RP, RP)])
        plsc.subcore_barrier()

        ebase0 = c * E + s * eper

        def eblk(i, _):
            eb = ebase0 + i * EB
            pltpu.sync_copy(src_hbm.at[pl.ds(eb, EB)], srcb)
            pltpu.sync_copy(dst_hbm.at[pl.ds(eb, EB)], dstb)
            cp = pltpu.async_copy(hs_hbm.at[pl.ds(tb, NP)].at[srcb], rows_v, sem)
            exs_list = []
            for j in range(EB // 16):
                sl = pl.ds(j * 16, 16)
                sv = srcb[sl]
                dv = dstb[sl]
                av = plsc.load_gather(als_v, [sv])
                ad = plsc.load_gather(ald_v, [dv])
                e = av + ad
                e = jnp.where(e >= 0.0, e, 0.2 * e)
                ex = jnp.exp(e)
                exs_list.append(ex)
                exb[sl] = ex
            cp.wait()
            for j in range(EB // 16):
                for el in range(16):
                    lid = jnp.full((16,), el, jnp.int32)
                    exs = _vgather(exs_list[j], lid)
                    for q in range(4):
                        qsl = pl.ds(q * 16, 16)
                        rows_v[j * 16 + el, qsl] = rows_v[j * 16 + el, qsl] * exs
            pltpu.sync_copy(rows_v, num_sh.at[dstb], add=True)
            pltpu.sync_copy(exb, den_sh.at[dstb], add=True)
            return ()
        lax.fori_loop(0, eper // EB, eblk, ())
        plsc.subcore_barrier()

        rb = s * RP
        pltpu.sync_copy(num_sh.at[pl.ds(rb, RP)], acc_v)
        pltpu.sync_copy(den_sh.at[pl.ds(rb, RP)], denb_v)
        bvs =[bias_v[pl.ds(q * 16, 16)] for q in range(4)]

        def drow(rc, _):
            for rr in range(16):
                r = rc * 16 + rr
                dsp = plsc.load_gather(denb_v, [jnp.broadcast_to(r, (16,))])
                dsp = dsp + 1e-16
                for q in range(4):
                    qsl = pl.ds(q * 16, 16)
                    acc_v[r, qsl] = acc_v[r, qsl] / dsp + bvs[q]
            return ()
        lax.fori_loop(0, RP // 16, drow, ())
        pltpu.sync_copy(acc_v, out_hbm.at[pl.ds(tb + rb, RP)])
        plsc.subcore_barrier()
        return ()

    lax.fori_loop(0, H, phase, ())


def _sc_gat(H, src_all, dst_all, als_flat, ald_flat, hs_flat, b_flat):
    mesh = plsc.VectorSubcoreMesh(core_axis_name="c", subcore_axis_name="s")
    f = pl.kernel(
        functools.partial(_sc_gat_body, H),
        out_type=jax.ShapeDtypeStruct((2 * H * NP, HID), jnp.float32),
        mesh=mesh,
        compiler_params=pltpu.CompilerParams(
            needs_layout_passes=False, use_tc_tiling_on_sc=False),
        scratch_types=[
            pltpu.VMEM((NP,), jnp.float32),
            pltpu.VMEM((NP,), jnp.float32),
            pltpu.VMEM((EB,), jnp.int32),
            pltpu.VMEM((EB,), jnp.int32),
            pltpu.VMEM((EB,), jnp.float32),
            pltpu.VMEM((EB, HID), jnp.float32),
            pltpu.VMEM((RP, HID), jnp.float32),
            pltpu.VMEM((RP,), jnp.float32),
            pltpu.VMEM((HID,), jnp.float32),
            pltpu.VMEM_SHARED((NP, HID), jnp.float32),
            pltpu.VMEM_SHARED((NP,), jnp.float32),
            pltpu.SemaphoreType.DMA,
        ],
    )
    return f(src_all, dst_all, als_flat, ald_flat, hs_flat, b_flat)


def kernel(x_author, x_paper, ei_writes, ei_written_by,
           w1s_wr, w1d_wr, a1s_wr, a1d_wr, b1_wr,
           w1s_wb, w1d_wb, a1s_wb, a1d_wb, b1_wb,
           w2_wr, a2s_wr, a2d_wr, b2_wr,
           w2_wb, a2s_wb, a2d_wb, b2_wb,
           lin_w, lin_b):
    f32 = jnp.float32
    x_pack = jnp.zeros((2, NP, D_IN), f32)
    x_pack = x_pack.at[0, :N].set(x_author).at[1, :N].set(x_paper)
    ws_pack = jnp.stack([w1s_wr, w1s_wb])
    as_pack = jnp.stack([a1s_wr, a1s_wb])
    wd_pack = jnp.stack([w1d_wb, w1d_wr])
    ad_pack = jnp.stack([a1d_wb, a1d_wr])
    src_all = jnp.concatenate([ei_writes[0], ei_written_by[0]])
    dst_all = jnp.concatenate([ei_writes[1], ei_written_by[1]])
    b1_flat = jnp.concatenate([b1_wr, b1_wb])
    b2_flat = jnp.concatenate([b2_wr, b2_wb])
    w2_pack = jnp.stack([w2_wr, w2_wb])
    a2s_pack = jnp.stack([a2s_wr, a2s_wb])
    a2d_pack = jnp.stack([a2d_wr, a2d_wb])

    hs_all, als_all, ald_all = _tc1(x_pack, ws_pack, as_pack, wd_pack, ad_pack)
    out1 = _sc_gat(HEADS, src_all, dst_all,
                   als_all.reshape(-1), ald_all.reshape(-1),
                   hs_all.reshape(-1, HID), b1_flat)
    out1 = out1.reshape(2, HEADS, NP, HID)

    hs2, als2, ald2 = _tc2(out1, w2_pack, a2s_pack, a2d_pack)
    out2 = _sc_gat(1, src_all, dst_all,
                   als2.reshape(-1), ald2.reshape(-1),
                   hs2.reshape(-1, HID), b2_flat)

    res = _tc3(out2.reshape(2, NP, HID), lin_w, lin_b)
    return (res[1, :N], res[0, :N])


# speedup vs baseline: 20.3135x; 1.2875x over previous
import functools

import jax
import jax.numpy as jnp
from jax import lax
from jax.experimental import pallas as pl
from jax.experimental.pallas import tpu as pltpu
from jax.experimental.pallas import tpu_sc as plsc

N = 10000
NP = 10240
E = 160000
D_IN = 256
HID = 64
HEADS = 8
OUT = 256

BN = 1024
NB = NP // BN
EB = 80
RP = NP // 16


def _tc1_body(x_ref, ws_ref, as_ref, wd_ref, ad_ref, hs_ref, als_ref, ald_ref):
    x = x_ref[0]
    for h in range(HEADS):
        ws_h = ws_ref[0, :, h * HID:(h + 1) * HID]
        yh = jnp.dot(x, ws_h, preferred_element_type=jnp.float32)
        hs_ref[0, h] = yh
        als_ref[0, h] = (yh * as_ref[0, h][None, :]).sum(-1)
        wd_h = wd_ref[0, :, h * HID:(h + 1) * HID]
        hdh = jnp.dot(x, wd_h, preferred_element_type=jnp.float32)
        ald_ref[0, h] = (hdh * ad_ref[0, h][None, :]).sum(-1)


def _tc1(x_pack, ws_pack, as_pack, wd_pack, ad_pack):
    return pl.pallas_call(
        _tc1_body,
        grid=(2, NB),
        in_specs=[
            pl.BlockSpec((1, BN, D_IN), lambda t, nb: (t, nb, 0)),
            pl.BlockSpec((1, D_IN, HEADS * HID), lambda t, nb: (t, 0, 0)),
            pl.BlockSpec((1, HEADS, HID), lambda t, nb: (t, 0, 0)),
            pl.BlockSpec((1, D_IN, HEADS * HID), lambda t, nb: (t, 0, 0)),
            pl.BlockSpec((1, HEADS, HID), lambda t, nb: (t, 0, 0)),
        ],
        out_specs=[
            pl.BlockSpec((1, HEADS, BN, HID), lambda t, nb: (t, 0, nb, 0)),
            pl.BlockSpec((1, HEADS, BN), lambda t, nb: (t, 0, nb)),
            pl.BlockSpec((1, HEADS, BN), lambda t, nb: (1 - t, 0, nb)),
        ],
        out_shape=[
            jax.ShapeDtypeStruct((2, HEADS, NP, HID), jnp.float32),
            jax.ShapeDtypeStruct((2, HEADS, NP), jnp.float32),
            jax.ShapeDtypeStruct((2, HEADS, NP), jnp.float32),
        ],
    )(x_pack, ws_pack, as_pack, wd_pack, ad_pack)


def _tc2_body(x_ref, w2a_ref, w2b_ref, a2s_ref, a2d_ref,
              hs2_ref, als2_ref, ald2_ref):
    hs2 = jnp.zeros((BN, HID), jnp.float32)
    hd2 = jnp.zeros((BN, HID), jnp.float32)
    for h in range(HEADS):
        xh = x_ref[0, h]
        hs2 = hs2 + jnp.dot(xh, w2a_ref[0, h * HID:(h + 1) * HID, :],
                            preferred_element_type=jnp.float32)
        hd2 = hd2 + jnp.dot(xh, w2b_ref[0, h * HID:(h + 1) * HID, :],
                            preferred_element_type=jnp.float32)
    hs2_ref[0] = hs2
    als2_ref[0, 0] = (hs2 * a2s_ref[0, 0][None, :]).sum(-1)
    ald2_ref[0, 0] = (hd2 * a2d_ref[0, 0][None, :]).sum(-1)


def _tc2(out1, w2_pack, a2s_pack, a2d_pack):
    return pl.pallas_call(
        _tc2_body,
        grid=(2, NB),
        in_specs=[
            pl.BlockSpec((1, HEADS, BN, HID), lambda u, nb: (1 - u, 0, nb, 0)),
            pl.BlockSpec((1, HEADS * HID, HID), lambda u, nb: (u, 0, 0)),
            pl.BlockSpec((1, HEADS * HID, HID), lambda u, nb: (1 - u, 0, 0)),
            pl.BlockSpec((1, 1, HID), lambda u, nb: (u, 0, 0)),
            pl.BlockSpec((1, 1, HID), lambda u, nb: (1 - u, 0, 0)),
        ],
        out_specs=[
            pl.BlockSpec((1, BN, HID), lambda u, nb: (u, nb, 0)),
            pl.BlockSpec((1, 1, BN), lambda u, nb: (u, 0, nb)),
            pl.BlockSpec((1, 1, BN), lambda u, nb: (1 - u, 0, nb)),
        ],
        out_shape=[
            jax.ShapeDtypeStruct((2, NP, HID), jnp.float32),
            jax.ShapeDtypeStruct((2, 1, NP), jnp.float32),
            jax.ShapeDtypeStruct((2, 1, NP), jnp.float32),
        ],
    )(out1, w2_pack, w2_pack, a2s_pack, a2d_pack)


def _tc3_body(x_ref, w_ref, b_ref, o_ref):
    o_ref[0] = (jnp.dot(x_ref[0], w_ref[...], preferred_element_type=jnp.float32)
                + b_ref[...][None, :])


def _tc3(out2, lin_w, lin_b):
    return pl.pallas_call(
        _tc3_body,
        grid=(2, NB),
        in_specs=[
            pl.BlockSpec((1, BN, HID), lambda t, nb: (t, nb, 0)),
            pl.BlockSpec((HID, OUT), lambda t, nb: (0, 0)),
            pl.BlockSpec((OUT,), lambda t, nb: (0,)),
        ],
        out_specs=pl.BlockSpec((1, BN, OUT), lambda t, nb: (t, nb, 0)),
        out_shape=jax.ShapeDtypeStruct((2, NP, OUT), jnp.float32),
    )(out2, lin_w, lin_b)


def _vgather(x, idx16):
    dnums = lax.GatherDimensionNumbers(
        offset_dims=(), collapsed_slice_dims=(0,), start_index_map=(0,))
    return lax.gather(x, idx16[:, None], dnums, (1,),
                      mode=lax.GatherScatterMode.PROMISE_IN_BOUNDS)


def _sc_gat_body(H, src_hbm, dst_hbm, als_hbm, ald_hbm, hs_hbm, b_hbm,
                 out_hbm, als_v, ald_v, srcb, dstb, exb, rows_v,
                 acc_v, denb_v, bias_v, num_sh, den_sh,
                 sem_g0, sem_g1, sem_s0, sem_s1, sem_d0, sem_d1):
    c = lax.axis_index("c")
    s = lax.axis_index("s")
    zero16 = jnp.zeros((16,), jnp.float32)
    eper = E // 16
    sems_g = (sem_g0, sem_g1)
    sems_s = (sem_s0, sem_s1)
    sems_d = (sem_d0, sem_d1)

    def phase(h, _):
        tb = (c * H + h) * NP
        pltpu.sync_copy(als_hbm.at[pl.ds(tb, NP)], als_v)
        pltpu.sync_copy(ald_hbm.at[pl.ds(tb, NP)], ald_v)
        pltpu.sync_copy(b_hbm.at[pl.ds((c * H + h) * HID, HID)], bias_v)

        def zacc(r, _):
            for q in range(4):
                acc_v[r, pl.ds(q * 16, 16)] = zero16
            return ()
        lax.fori_loop(0, RP, zacc, ())

        def zden(i, _):
            denb_v[pl.ds(i * 16, 16)] = zero16
            return ()
        lax.fori_loop(0, RP // 16, zden, ())
        pltpu.sync_copy(acc_v, num_sh.at[pl.ds(s * RP, RP)])
        pltpu.sync_copy(denb_v, den_sh.at[pl.ds(s * RP, RP)])
        plsc.subcore_barrier()

        ebase0 = c * E + s * eper
        nblk = eper // EB

        def load_and_gather(p, blk):
            eb = ebase0 + blk * EB
            pltpu.sync_copy(src_hbm.at[pl.ds(eb, EB)], srcb.at[p])
            pltpu.sync_copy(dst_hbm.at[pl.ds(eb, EB)], dstb.at[p])
            return pltpu.async_copy(hs_hbm.at[pl.ds(tb, NP)].at[srcb.at[p]],
                                    rows_v.at[p], sems_g[p])

        def logits(p):
            exs_list = []
            for j in range(EB // 16):
                sl = pl.ds(j * 16, 16)
                sv = srcb[p, sl]
                dv = dstb[p, sl]
                av = plsc.load_gather(als_v, [sv])
                ad = plsc.load_gather(ald_v, [dv])
                e = av + ad
                e = jnp.where(e >= 0.0, e, 0.2 * e)
                ex = jnp.exp(e)
                exs_list.append(ex)
                exb[p, sl] = ex
            return exs_list

        def scale_and_scatter(p, exs_list):
            for j in range(EB // 16):
                for el in range(16):
                    lid = jnp.full((16,), el, jnp.int32)
                    exs = _vgather(exs_list[j], lid)
                    for q in range(4):
                        qsl = pl.ds(q * 16, 16)
                        rows_v[p, j * 16 + el, qsl] = rows_v[p, j * 16 + el, qsl] * exs
            cs = pltpu.async_copy(rows_v.at[p], num_sh.at[dstb.at[p]], sems_s[p],
                                  add=True)
            cd = pltpu.async_copy(exb.at[p], den_sh.at[dstb.at[p]], sems_d[p],
                                  add=True)
            return cs, cd

        def epair(i, _):
            g0 = load_and_gather(0, 2 * i)
            ex0 = logits(0)
            g1 = load_and_gather(1, 2 * i + 1)
            ex1 = logits(1)
            g0.wait()
            s0, d0 = scale_and_scatter(0, ex0)
            g1.wait()
            s1, d1 = scale_and_scatter(1, ex1)
            s0.wait()
            d0.wait()
            s1.wait()
            d1.wait()
            return ()
        lax.fori_loop(0, nblk // 2, epair, ())
        if nblk % 2:
            gt = load_and_gather(0, nblk - 1)
            ext = logits(0)
            gt.wait()
            st, dt = scale_and_scatter(0, ext)
            st.wait()
            dt.wait()
        plsc.subcore_barrier()

        rb = s * RP
        pltpu.sync_copy(num_sh.at[pl.ds(rb, RP)], acc_v)
        pltpu.sync_copy(den_sh.at[pl.ds(rb, RP)], denb_v)
        bvs =[bias_v[pl.ds(q * 16, 16)] for q in range(4)]

        def drow(rc, _):
            for rr in range(16):
                r = rc * 16 + rr
                dsp = plsc.load_gather(denb_v, [jnp.broadcast_to(r, (16,))])
                dsp = dsp + 1e-16
                for q in range(4):
                    qsl = pl.ds(q * 16, 16)
                    acc_v[r, qsl] = acc_v[r, qsl] / dsp + bvs[q]
            return ()
        lax.fori_loop(0, RP // 16, drow, ())
        pltpu.sync_copy(acc_v, out_hbm.at[pl.ds(tb + rb, RP)])
        plsc.subcore_barrier()
        return ()

    lax.fori_loop(0, H, phase, ())


def _sc_gat(H, src_all, dst_all, als_flat, ald_flat, hs_flat, b_flat):
    mesh = plsc.VectorSubcoreMesh(core_axis_name="c", subcore_axis_name="s")
    f = pl.kernel(
        functools.partial(_sc_gat_body, H),
        out_type=jax.ShapeDtypeStruct((2 * H * NP, HID), jnp.float32),
        mesh=mesh,
        compiler_params=pltpu.CompilerParams(
            needs_layout_passes=False, use_tc_tiling_on_sc=False),
        scratch_types=[
            pltpu.VMEM((NP,), jnp.float32),
            pltpu.VMEM((NP,), jnp.float32),
            pltpu.VMEM((2, EB), jnp.int32),
            pltpu.VMEM((2, EB), jnp.int32),
            pltpu.VMEM((2, EB), jnp.float32),
            pltpu.VMEM((2, EB, HID), jnp.float32),
            pltpu.VMEM((RP, HID), jnp.float32),
            pltpu.VMEM((RP,), jnp.float32),
            pltpu.VMEM((HID,), jnp.float32),
            pltpu.VMEM_SHARED((NP, HID), jnp.float32),
            pltpu.VMEM_SHARED((NP,), jnp.float32),
            pltpu.SemaphoreType.DMA,
            pltpu.SemaphoreType.DMA,
            pltpu.SemaphoreType.DMA,
            pltpu.SemaphoreType.DMA,
            pltpu.SemaphoreType.DMA,
            pltpu.SemaphoreType.DMA,
        ],
    )
    return f(src_all, dst_all, als_flat, ald_flat, hs_flat, b_flat)


def kernel(x_author, x_paper, ei_writes, ei_written_by,
           w1s_wr, w1d_wr, a1s_wr, a1d_wr, b1_wr,
           w1s_wb, w1d_wb, a1s_wb, a1d_wb, b1_wb,
           w2_wr, a2s_wr, a2d_wr, b2_wr,
           w2_wb, a2s_wb, a2d_wb, b2_wb,
           lin_w, lin_b):
    f32 = jnp.float32
    x_pack = jnp.zeros((2, NP, D_IN), f32)
    x_pack = x_pack.at[0, :N].set(x_author).at[1, :N].set(x_paper)
    ws_pack = jnp.stack([w1s_wr, w1s_wb])
    as_pack = jnp.stack([a1s_wr, a1s_wb])
    wd_pack = jnp.stack([w1d_wb, w1d_wr])
    ad_pack = jnp.stack([a1d_wb, a1d_wr])
    src_all = jnp.concatenate([ei_writes[0], ei_written_by[0]])
    dst_all = jnp.concatenate([ei_writes[1], ei_written_by[1]])
    b1_flat = jnp.concatenate([b1_wr, b1_wb])
    b2_flat = jnp.concatenate([b2_wr, b2_wb])
    w2_pack = jnp.stack([w2_wr, w2_wb])
    a2s_pack = jnp.stack([a2s_wr, a2s_wb])
    a2d_pack = jnp.stack([a2d_wr, a2d_wb])

    hs_all, als_all, ald_all = _tc1(x_pack, ws_pack, as_pack, wd_pack, ad_pack)
    out1 = _sc_gat(HEADS, src_all, dst_all,
                   als_all.reshape(-1), ald_all.reshape(-1),
                   hs_all.reshape(-1, HID), b1_flat)
    out1 = out1.reshape(2, HEADS, NP, HID)

    hs2, als2, ald2 = _tc2(out1, w2_pack, a2s_pack, a2d_pack)
    out2 = _sc_gat(1, src_all, dst_all,
                   als2.reshape(-1), ald2.reshape(-1),
                   hs2.reshape(-1, HID), b2_flat)

    res = _tc3(out2.reshape(2, NP, HID), lin_w, lin_b)
    return (res[1, :N], res[0, :N])


# speedup vs baseline: 20.7042x; 1.0192x over previous
import functools

import jax
import jax.numpy as jnp
from jax import lax
from jax.experimental import pallas as pl
from jax.experimental.pallas import tpu as pltpu
from jax.experimental.pallas import tpu_sc as plsc

N = 10000
NP = 10240
E = 160000
D_IN = 256
HID = 64
HEADS = 8
OUT = 256

BN = 1024
NB = NP // BN
EB = 80
RP = NP // 16


def _tc1_body(x_ref, ws_ref, as_ref, wd_ref, ad_ref, hs_ref, als_ref, ald_ref):
    x = x_ref[0]
    for h in range(HEADS):
        ws_h = ws_ref[0, :, h * HID:(h + 1) * HID]
        yh = jnp.dot(x, ws_h, preferred_element_type=jnp.float32)
        hs_ref[0, h] = yh
        als_ref[0, h] = (yh * as_ref[0, h][None, :]).sum(-1)
        wd_h = wd_ref[0, :, h * HID:(h + 1) * HID]
        hdh = jnp.dot(x, wd_h, preferred_element_type=jnp.float32)
        ald_ref[0, h] = (hdh * ad_ref[0, h][None, :]).sum(-1)


def _tc1(x_pack, ws_pack, as_pack, wd_pack, ad_pack):
    return pl.pallas_call(
        _tc1_body,
        grid=(2, NB),
        in_specs=[
            pl.BlockSpec((1, BN, D_IN), lambda t, nb: (t, nb, 0)),
            pl.BlockSpec((1, D_IN, HEADS * HID), lambda t, nb: (t, 0, 0)),
            pl.BlockSpec((1, HEADS, HID), lambda t, nb: (t, 0, 0)),
            pl.BlockSpec((1, D_IN, HEADS * HID), lambda t, nb: (t, 0, 0)),
            pl.BlockSpec((1, HEADS, HID), lambda t, nb: (t, 0, 0)),
        ],
        out_specs=[
            pl.BlockSpec((1, HEADS, BN, HID), lambda t, nb: (t, 0, nb, 0)),
            pl.BlockSpec((1, HEADS, BN), lambda t, nb: (t, 0, nb)),
            pl.BlockSpec((1, HEADS, BN), lambda t, nb: (1 - t, 0, nb)),
        ],
        out_shape=[
            jax.ShapeDtypeStruct((2, HEADS, NP, HID), jnp.float32),
            jax.ShapeDtypeStruct((2, HEADS, NP), jnp.float32),
            jax.ShapeDtypeStruct((2, HEADS, NP), jnp.float32),
        ],
    )(x_pack, ws_pack, as_pack, wd_pack, ad_pack)


def _tc2_body(x_ref, w2a_ref, w2b_ref, a2s_ref, a2d_ref,
              hs2_ref, als2_ref, ald2_ref):
    hs2 = jnp.zeros((BN, HID), jnp.float32)
    hd2 = jnp.zeros((BN, HID), jnp.float32)
    for h in range(HEADS):
        xh = x_ref[0, h]
        hs2 = hs2 + jnp.dot(xh, w2a_ref[0, h * HID:(h + 1) * HID, :],
                            preferred_element_type=jnp.float32)
        hd2 = hd2 + jnp.dot(xh, w2b_ref[0, h * HID:(h + 1) * HID, :],
                            preferred_element_type=jnp.float32)
    hs2_ref[0] = hs2
    als2_ref[0, 0] = (hs2 * a2s_ref[0, 0][None, :]).sum(-1)
    ald2_ref[0, 0] = (hd2 * a2d_ref[0, 0][None, :]).sum(-1)


def _tc2(out1, w2_pack, a2s_pack, a2d_pack):
    return pl.pallas_call(
        _tc2_body,
        grid=(2, NB),
        in_specs=[
            pl.BlockSpec((1, HEADS, BN, HID), lambda u, nb: (1 - u, 0, nb, 0)),
            pl.BlockSpec((1, HEADS * HID, HID), lambda u, nb: (u, 0, 0)),
            pl.BlockSpec((1, HEADS * HID, HID), lambda u, nb: (1 - u, 0, 0)),
            pl.BlockSpec((1, 1, HID), lambda u, nb: (u, 0, 0)),
            pl.BlockSpec((1, 1, HID), lambda u, nb: (1 - u, 0, 0)),
        ],
        out_specs=[
            pl.BlockSpec((1, BN, HID), lambda u, nb: (u, nb, 0)),
            pl.BlockSpec((1, 1, BN), lambda u, nb: (u, 0, nb)),
            pl.BlockSpec((1, 1, BN), lambda u, nb: (1 - u, 0, nb)),
        ],
        out_shape=[
            jax.ShapeDtypeStruct((2, NP, HID), jnp.float32),
            jax.ShapeDtypeStruct((2, 1, NP), jnp.float32),
            jax.ShapeDtypeStruct((2, 1, NP), jnp.float32),
        ],
    )(out1, w2_pack, w2_pack, a2s_pack, a2d_pack)


def _tc3_body(x_ref, w_ref, b_ref, o_ref):
    o_ref[0] = (jnp.dot(x_ref[0], w_ref[...], preferred_element_type=jnp.float32)
                + b_ref[...][None, :])


def _tc3(out2, lin_w, lin_b):
    return pl.pallas_call(
        _tc3_body,
        grid=(2, NB),
        in_specs=[
            pl.BlockSpec((1, BN, HID), lambda t, nb: (t, nb, 0)),
            pl.BlockSpec((HID, OUT), lambda t, nb: (0, 0)),
            pl.BlockSpec((OUT,), lambda t, nb: (0,)),
        ],
        out_specs=pl.BlockSpec((1, BN, OUT), lambda t, nb: (t, nb, 0)),
        out_shape=jax.ShapeDtypeStruct((2, NP, OUT), jnp.float32),
    )(out2, lin_w, lin_b)


def _vgather(x, idx16):
    dnums = lax.GatherDimensionNumbers(
        offset_dims=(), collapsed_slice_dims=(0,), start_index_map=(0,))
    return lax.gather(x, idx16[:, None], dnums, (1,),
                      mode=lax.GatherScatterMode.PROMISE_IN_BOUNDS)


def _sc_gat_body(H, src_hbm, dst_hbm, als_hbm, ald_hbm, hs_hbm, b_hbm,
                 out_hbm, als_v, ald_v, srcb, dstb, exb, rows_v,
                 acc_v, denb_v, bias_v, num_sh, den_sh,
                 sem_g0, sem_g1, sem_g2, sem_g3,
                 sem_s0, sem_s1, sem_s2, sem_s3,
                 sem_d0, sem_d1, sem_d2, sem_d3):
    c = lax.axis_index("c")
    s = lax.axis_index("s")
    zero16 = jnp.zeros((16,), jnp.float32)
    eper = E // 16
    sems_g = (sem_g0, sem_g1, sem_g2, sem_g3)
    sems_s = (sem_s0, sem_s1, sem_s2, sem_s3)
    sems_d = (sem_d0, sem_d1, sem_d2, sem_d3)

    def phase(h, _):
        tb = (c * H + h) * NP
        pltpu.sync_copy(als_hbm.at[pl.ds(tb, NP)], als_v)
        pltpu.sync_copy(ald_hbm.at[pl.ds(tb, NP)], ald_v)
        pltpu.sync_copy(b_hbm.at[pl.ds((c * H + h) * HID, HID)], bias_v)

        def zacc(r, _):
            for q in range(4):
                acc_v[r, pl.ds(q * 16, 16)] = zero16
            return ()
        lax.fori_loop(0, RP, zacc, ())

        def zden(i, _):
            denb_v[pl.ds(i * 16, 16)] = zero16
            return ()
        lax.fori_loop(0, RP // 16, zden, ())
        pltpu.sync_copy(acc_v, num_sh.at[pl.ds(s * RP, RP)])
        pltpu.sync_copy(denb_v, den_sh.at[pl.ds(s * RP, RP)])
        plsc.subcore_barrier()

        ebase0 = c * E + s * eper
        nblk = eper // EB

        def load_and_gather(p, blk):
            eb = ebase0 + blk * EB
            pltpu.sync_copy(src_hbm.at[pl.ds(eb, EB)], srcb.at[p])
            pltpu.sync_copy(dst_hbm.at[pl.ds(eb, EB)], dstb.at[p])
            return pltpu.async_copy(hs_hbm.at[pl.ds(tb, NP)].at[srcb.at[p]],
                                    rows_v.at[p], sems_g[p])

        def logits(p):
            exs_list = []
            for j in range(EB // 16):
                sl = pl.ds(j * 16, 16)
                sv = srcb[p, sl]
                dv = dstb[p, sl]
                av = plsc.load_gather(als_v, [sv])
                ad = plsc.load_gather(ald_v, [dv])
                e = av + ad
                e = jnp.where(e >= 0.0, e, 0.2 * e)
                ex = jnp.exp(e)
                exs_list.append(ex)
                exb[p, sl] = ex
            return exs_list

        def scale_and_scatter(p, exs_list):
            for j in range(EB // 16):
                for el in range(16):
                    lid = jnp.full((16,), el, jnp.int32)
                    exs = _vgather(exs_list[j], lid)
                    for q in range(4):
                        qsl = pl.ds(q * 16, 16)
                        rows_v[p, j * 16 + el, qsl] = rows_v[p, j * 16 + el, qsl] * exs
            cs = pltpu.async_copy(rows_v.at[p], num_sh.at[dstb.at[p]], sems_s[p],
                                  add=True)
            cd = pltpu.async_copy(exb.at[p], den_sh.at[dstb.at[p]], sems_d[p],
                                  add=True)
            return cs, cd

        NBUF = 4

        def equad(i, _):
            gs = []
            exs = []
            for p in range(NBUF):
                gs.append(load_and_gather(p, NBUF * i + p))
                exs.append(logits(p))
            waits = []
            for p in range(NBUF):
                gs[p].wait()
                waits.append(scale_and_scatter(p, exs[p]))
            for sp, dp in waits:
                sp.wait()
                dp.wait()
            return ()
        lax.fori_loop(0, nblk // NBUF, equad, ())
        for r in range(nblk % NBUF):
            gt = load_and_gather(0, nblk - (nblk % NBUF) + r)
            ext = logits(0)
            gt.wait()
            st, dt = scale_and_scatter(0, ext)
            st.wait()
            dt.wait()
        plsc.subcore_barrier()

        rb = s * RP
        pltpu.sync_copy(num_sh.at[pl.ds(rb, RP)], acc_v)
        pltpu.sync_copy(den_sh.at[pl.ds(rb, RP)], denb_v)
        bvs =[bias_v[pl.ds(q * 16, 16)] for q in range(4)]

        def drow(rc, _):
            for rr in range(16):
                r = rc * 16 + rr
                dsp = plsc.load_gather(denb_v, [jnp.broadcast_to(r, (16,))])
                dsp = dsp + 1e-16
                for q in range(4):
                    qsl = pl.ds(q * 16, 16)
                    acc_v[r, qsl] = acc_v[r, qsl] / dsp + bvs[q]
            return ()
        lax.fori_loop(0, RP // 16, drow, ())
        pltpu.sync_copy(acc_v, out_hbm.at[pl.ds(tb + rb, RP)])
        plsc.subcore_barrier()
        return ()

    lax.fori_loop(0, H, phase, ())


def _sc_gat(H, src_all, dst_all, als_flat, ald_flat, hs_flat, b_flat):
    mesh = plsc.VectorSubcoreMesh(core_axis_name="c", subcore_axis_name="s")
    f = pl.kernel(
        functools.partial(_sc_gat_body, H),
        out_type=jax.ShapeDtypeStruct((2 * H * NP, HID), jnp.float32),
        mesh=mesh,
        compiler_params=pltpu.CompilerParams(
            needs_layout_passes=False, use_tc_tiling_on_sc=False),
        scratch_types=[
            pltpu.VMEM((NP,), jnp.float32),
            pltpu.VMEM((NP,), jnp.float32),
            pltpu.VMEM((4, EB), jnp.int32),
            pltpu.VMEM((4, EB), jnp.int32),
            pltpu.VMEM((4, EB), jnp.float32),
            pltpu.VMEM((4, EB, HID), jnp.float32),
            pltpu.VMEM((RP, HID), jnp.float32),
            pltpu.VMEM((RP,), jnp.float32),
            pltpu.VMEM((HID,), jnp.float32),
            pltpu.VMEM_SHARED((NP, HID), jnp.float32),
            pltpu.VMEM_SHARED((NP,), jnp.float32),
        ] + [pltpu.SemaphoreType.DMA] * 12,
    )
    return f(src_all, dst_all, als_flat, ald_flat, hs_flat, b_flat)


def kernel(x_author, x_paper, ei_writes, ei_written_by,
           w1s_wr, w1d_wr, a1s_wr, a1d_wr, b1_wr,
           w1s_wb, w1d_wb, a1s_wb, a1d_wb, b1_wb,
           w2_wr, a2s_wr, a2d_wr, b2_wr,
           w2_wb, a2s_wb, a2d_wb, b2_wb,
           lin_w, lin_b):
    f32 = jnp.float32
    x_pack = jnp.zeros((2, NP, D_IN), f32)
    x_pack = x_pack.at[0, :N].set(x_author).at[1, :N].set(x_paper)
    ws_pack = jnp.stack([w1s_wr, w1s_wb])
    as_pack = jnp.stack([a1s_wr, a1s_wb])
    wd_pack = jnp.stack([w1d_wb, w1d_wr])
    ad_pack = jnp.stack([a1d_wb, a1d_wr])
    src_all = jnp.concatenate([ei_writes[0], ei_written_by[0]])
    dst_all = jnp.concatenate([ei_writes[1], ei_written_by[1]])
    b1_flat = jnp.concatenate([b1_wr, b1_wb])
    b2_flat = jnp.concatenate([b2_wr, b2_wb])
    w2_pack = jnp.stack([w2_wr, w2_wb])
    a2s_pack = jnp.stack([a2s_wr, a2s_wb])
    a2d_pack = jnp.stack([a2d_wr, a2d_wb])

    hs_all, als_all, ald_all = _tc1(x_pack, ws_pack, as_pack, wd_pack, ad_pack)
    out1 = _sc_gat(HEADS, src_all, dst_all,
                   als_all.reshape(-1), ald_all.reshape(-1),
                   hs_all.reshape(-1, HID), b1_flat)
    out1 = out1.reshape(2, HEADS, NP, HID)

    hs2, als2, ald2 = _tc2(out1, w2_pack, a2s_pack, a2d_pack)
    out2 = _sc_gat(1, src_all, dst_all,
                   als2.reshape(-1), ald2.reshape(-1),
                   hs2.reshape(-1, HID), b2_flat)

    res = _tc3(out2.reshape(2, NP, HID), lin_w, lin_b)
    return (res[1, :N], res[0, :N])
